# Initial kernel scaffold; baseline (speedup 1.0000x reference)
import jax
import jax.numpy as jnp
from jax.experimental import pallas as pl


def kernel(x, edge_index, batch, W0, b0, W1, b1, W2, b2, Wm, bm):
    raise NotImplementedError("write your pallas kernel here")


# speedup vs baseline: 5.7681x; 5.7681x over previous
import functools

import jax
import jax.numpy as jnp
from jax import lax
from jax.experimental import pallas as pl
from jax.experimental.pallas import tpu as pltpu
from jax.experimental.pallas import tpu_sc as plsc

N = 10000
E = 320000
F = 128
H = 128
G = 64
C = 2

NPAD = 10240
DUMMY = NPAD - 1
EPAD = 327680
NCHUNKS = EPAD // 128
NTILES = 32
CPT = NCHUNKS // NTILES
RPT = NPAD // 16
GRP = 2
IB = 16


def _sc_mesh():
    return plsc.VectorSubcoreMesh(
        core_axis_name="c", subcore_axis_name="s", num_cores=2, num_subcores=16
    )


def _make_degree(width):
    @functools.partial(
        pl.kernel,
        out_type=jax.ShapeDtypeStruct((2, NPAD, width), jnp.float32),
        mesh=_sc_mesh(),
        scratch_types=[
            pltpu.VMEM((CPT, 128), jnp.int32),
            pltpu.VMEM((128, width), jnp.float32),
            pltpu.VMEM_SHARED((NPAD, width), jnp.float32),
        ],
    )
    def degree(dsts, ones_hbm, zeros, out, dst_v, ones_v, accum):
        c = lax.axis_index("c")
        s = lax.axis_index("s")
        wid = s * 2 + c
        pltpu.sync_copy(dsts.at[pl.ds(wid * CPT, CPT)], dst_v)
        pltpu.sync_copy(ones_hbm, ones_v)
        pltpu.sync_copy(zeros, accum.at[pl.ds(s * RPT, RPT)])
        plsc.subcore_barrier()

        def step(j, carry):
            pltpu.sync_copy(ones_v, accum.at[dst_v.at[j]], add=True)
            return carry

        lax.fori_loop(0, CPT, step, 0)
        plsc.subcore_barrier()
        pltpu.sync_copy(
            accum.at[pl.ds(s * RPT, RPT)], out.at[c, pl.ds(s * RPT, RPT)]
        )

    return degree


def _make_prop(width):
    mesh = _sc_mesh()

    @functools.partial(
        pl.kernel,
        out_type=jax.ShapeDtypeStruct((2, NPAD, width), jnp.float32),
        mesh=mesh,
        scratch_types=[
            pltpu.VMEM((IB, 128), jnp.int32),
            pltpu.VMEM((IB, 128), jnp.int32),
            pltpu.VMEM((128, width), jnp.float32),
            pltpu.VMEM((128, width), jnp.float32),
            pltpu.VMEM_SHARED((NPAD, width), jnp.float32),
            pltpu.SemaphoreType.DMA,
            pltpu.SemaphoreType.DMA,
        ],
    )
    def prop(table, srcs, dsts, zeros, out, src_v, dst_v, buf0, buf1, accum, sem0, sem1):
        c = lax.axis_index("c")
        s = lax.axis_index("s")
        wid = s * 2 + c
        pltpu.sync_copy(zeros, accum.at[pl.ds(s * RPT, RPT)])
        plsc.subcore_barrier()

        bufs = (buf0, buf1)
        sems = (sem0, sem1)

        def block(ib, carry):
            base = wid * CPT + ib * IB
            pltpu.sync_copy(srcs.at[pl.ds(base, IB)], src_v)
            pltpu.sync_copy(dsts.at[pl.ds(base, IB)], dst_v)

            def outer(jo, carry2):
                descs = []
                for b in range(GRP):
                    j = jo * GRP + b
                    descs.append(
                        pltpu.async_copy(table.at[src_v.at[j]], bufs[b], sems[b])
                    )
                for b in range(GRP):
                    j = jo * GRP + b
                    descs[b].wait()
                    pltpu.sync_copy(bufs[b], accum.at[dst_v.at[j]], add=True)
                return carry2

            lax.fori_loop(0, IB // GRP, outer, 0)
            return carry

        lax.fori_loop(0, CPT // IB, block, 0)
        plsc.subcore_barrier()
        pltpu.sync_copy(
            accum.at[pl.ds(s * RPT, RPT)], out.at[c, pl.ds(s * RPT, RPT)]
        )

    return prop


_degree128 = _make_degree(128)
_prop128 = _make_prop(128)


def _prep_body(degp0, degp1, x_ref, norm_ref, hn_ref):
    deg = degp0[:, 0:1] + degp1[:, 0:1] + 1.0
    norm = lax.rsqrt(deg)
    norm_ref[...] = norm
    hn_ref[...] = x_ref[...] * norm


def _prep(degp0, degp1, x_p):
    return pl.pallas_call(
        _prep_body,
        out_shape=(
            jax.ShapeDtypeStruct((NPAD, 1), jnp.float32),
            jax.ShapeDtypeStruct((NPAD, H), jnp.float32),
        ),
    )(degp0, degp1, x_p)


def _layer_body(p0, p1, hn, norm, W, b, out):
    agg = (p0[...] + p1[...] + hn[...]) * norm[...]
    h = jnp.maximum(
        jnp.dot(agg, W[...], preferred_element_type=jnp.float32) + b[...], 0.0
    )
    out[...] = h * norm[...]


def _layer(p0, p1, hn, norm, W, b):
    return pl.pallas_call(
        _layer_body,
        out_shape=jax.ShapeDtypeStruct((NPAD, H), jnp.float32),
    )(p0, p1, hn, norm, W, b)


NSCAN = 14


def _final_body(
    p0, p1, hn, norm, W, b, batch_ref, same_ref, end_ref, Wm1, Wm2, bm,
    out, h_scr, pa, pb,
):
    agg = (p0[...] + p1[...] + hn[...]) * norm[...]
    h_scr[...] = jnp.maximum(
        jnp.dot(agg, W[...], preferred_element_type=jnp.float32) + b[...], 0.0
    )

    bufs = (pa, pb)
    src = h_scr
    for k in range(NSCAN):
        dst = bufs[k % 2]
        d = 1 << k
        L = NPAD - d
        dst[pl.ds(0, d), :] = src[pl.ds(0, d), :]
        shifted = src[pl.ds(0, L), :]
        cur = src[pl.ds(d, L), :]
        same = same_ref[pl.ds(d, L), k:k + 1]
        dst[pl.ds(d, L), :] = jnp.maximum(
            cur, jnp.where(same > 0.0, shifted, -1e30)
        )
        src = dst
    pref = src

    ones = jnp.ones((128, 128), jnp.float32)

    def chunk(ci, carry):
        ms, mx, cnt = carry
        hc = h_scr[pl.ds(ci * 128, 128), :]
        pc = pref[pl.ds(ci * 128, 128), :]
        bc = batch_ref[0:1, pl.ds(ci * 128, 128)]
        ec = end_ref[0:1, pl.ds(ci * 128, 128)]
        ids = lax.broadcasted_iota(jnp.int32, (G, 128), 0)
        eqf = (ids == bc).astype(jnp.float32)
        ms = ms + jnp.dot(eqf, hc, preferred_element_type=jnp.float32)
        cnt = cnt + jnp.dot(eqf, ones, preferred_element_type=jnp.float32)
        mx = mx + jnp.dot(eqf * ec, pc, preferred_element_type=jnp.float32)
        return ms, mx, cnt

    init = (
        jnp.zeros((G, H), jnp.float32),
        jnp.zeros((G, H), jnp.float32),
        jnp.zeros((G, H), jnp.float32),
    )
    ms, mx, cnt = lax.fori_loop(0, NPAD // 128, chunk, init)
    meanp = ms / jnp.maximum(cnt, 1.0)
    logits = (
        jnp.dot(meanp, Wm1[...], preferred_element_type=jnp.float32)
        + jnp.dot(mx, Wm2[...], preferred_element_type=jnp.float32)
        + bm[...]
    )
    out[...] = jax.nn.sigmoid(logits)


def _final(p0, p1, hn, norm, W, b, batch_p, same_m, end_m, Wm1, Wm2, bm):
    return pl.pallas_call(
        _final_body,
        out_shape=jax.ShapeDtypeStruct((G, C), jnp.float32),
        scratch_shapes=[
            pltpu.VMEM((NPAD, H), jnp.float32),
            pltpu.VMEM((NPAD, H), jnp.float32),
            pltpu.VMEM((NPAD, H), jnp.float32),
        ],
    )(p0, p1, hn, norm, W, b, batch_p, same_m, end_m, Wm1, Wm2, bm)


def kernel(x, edge_index, batch, W0, b0, W1, b1, W2, b2, Wm, bm):
    src = edge_index[0]
    dst = edge_index[1]
    pad_e = EPAD - E
    src_p = jnp.concatenate([src, jnp.zeros((pad_e,), jnp.int32)]).reshape(NCHUNKS, 128)
    dst_p = jnp.concatenate(
        [dst, jnp.full((pad_e,), DUMMY, jnp.int32)]
    ).reshape(NCHUNKS, 128)
    x_p = jnp.pad(x, ((0, NPAD - N), (0, 0)))
    batch_pad = jnp.pad(batch, (0, NPAD - N), constant_values=2**30)
    batch_p = batch_pad.reshape(1, NPAD)
    same_cols = [
        jnp.concatenate(
            [jnp.zeros((1 << k,), jnp.bool_), batch_pad[1 << k:] == batch_pad[:-(1 << k)]]
        )
        for k in range(NSCAN)
    ]
    same_m = jnp.stack(
        same_cols + [jnp.zeros((NPAD,), jnp.bool_)] * (16 - NSCAN), axis=1
    ).astype(jnp.float32)
    end_m = jnp.concatenate(
        [batch_pad[:-1] != batch_pad[1:], jnp.ones((1,), jnp.bool_)]
    ).astype(jnp.float32).reshape(1, NPAD)
    zeros_w = jnp.zeros((RPT, H), jnp.float32)
    ones_tab = jnp.ones((128, 128), jnp.float32)

    degp = _degree128(dst_p, ones_tab, zeros_w)
    norm, hn = _prep(degp[0], degp[1], x_p)
    p = _prop128(hn, src_p, dst_p, zeros_w)
    hn = _layer(p[0], p[1], hn, norm, W0, b0.reshape(1, H))
    p = _prop128(hn, src_p, dst_p, zeros_w)
    hn = _layer(p[0], p[1], hn, norm, W1, b1.reshape(1, H))
    p = _prop128(hn, src_p, dst_p, zeros_w)
    return _final(
        p[0], p[1], hn, norm, W2, b2.reshape(1, H),
        batch_p, same_m, end_m, Wm[:H], Wm[H:], bm.reshape(1, C),
    )
